# Initial kernel scaffold; baseline (speedup 1.0000x reference)
import functools

import jax
import jax.numpy as jnp
from jax import lax
from jax.experimental import pallas as pl
from jax.experimental.pallas import tpu as pltpu
from jax.experimental.pallas import tpu_sc as plsc

D = 256
L = 16
NC, NS = 2, 16
NW = NC * NS
W = 128


def _build_sc_kernel(E):
    n_win = E // W
    assert n_win * W == E
    steps = (n_win + NW - 1) // NW

    @functools.partial(
        pl.kernel,
        mesh=plsc.VectorSubcoreMesh(core_axis_name="c", subcore_axis_name="s"),
        out_type=jax.ShapeDtypeStruct((E,), jnp.float32),
        scratch_types=[
            pltpu.VMEM((W,), jnp.int32),
            pltpu.VMEM((W,), jnp.int32),
            pltpu.VMEM((W, D), jnp.float32),
            pltpu.VMEM((W, D), jnp.float32),
            pltpu.VMEM((W,), jnp.float32),
            pltpu.SemaphoreType.DMA,
            pltpu.SemaphoreType.DMA,
        ],
    )
    def k(x_hbm, src_hbm, dst_hbm, out_hbm,
          sidx, didx, srows, drows, outv, sem_s, sem_d):
        wid = lax.axis_index("s") * NC + lax.axis_index("c")

        @pl.loop(0, steps)
        def _(t):
            j = wid + t * NW

            @pl.when(j < n_win)
            def _():
                base = j * W
                pltpu.sync_copy(src_hbm.at[pl.ds(base, W)], sidx)
                pltpu.sync_copy(dst_hbm.at[pl.ds(base, W)], didx)
                cs = pltpu.async_copy(x_hbm.at[sidx], srows, sem_s)
                cd = pltpu.async_copy(x_hbm.at[didx], drows, sem_d)
                cs.wait()
                cd.wait()

                @pl.loop(0, W // L)
                def _(g):
                    lane = lax.broadcasted_iota(jnp.int32, (L,), 0)
                    res = jnp.zeros((L,), jnp.float32)
                    for i in range(L):
                        e = g * L + i
                        acc = srows[e, pl.ds(0, L)] * drows[e, pl.ds(0, L)]
                        for c in range(1, D // L):
                            acc = acc + (srows[e, pl.ds(c * L, L)]
                                         * drows[e, pl.ds(c * L, L)])
                        res = jnp.where(lane == i, jnp.sum(acc), res)
                    outv[pl.ds(g * L, L)] = res

                pltpu.sync_copy(outv, out_hbm.at[pl.ds(base, W)])

    return k


@jax.jit
def kernel(x, edge_index):
    src = edge_index[0].astype(jnp.int32)
    dst = edge_index[1].astype(jnp.int32)
    return _build_sc_kernel(src.shape[0])(x, src, dst)


# speedup vs baseline: 1.8363x; 1.8363x over previous
import dataclasses
import functools

import jax
import jax.numpy as jnp
from jax import lax
from jax.experimental import pallas as pl
from jax.experimental.pallas import tpu as pltpu
from jax.experimental.pallas import tpu_sc as plsc

D = 256
L = 16
NC, NS = 2, 16
NW = NC * NS
W = 128


def _build_sc_kernel(E):
    n_win = E // W
    assert n_win * W == E
    steps = (n_win + NW - 1) // NW

    cp = pltpu.CompilerParams()
    if "needs_layout_passes" in pltpu.CompilerParams.__dataclass_fields__:
        cp = dataclasses.replace(cp, needs_layout_passes=False)

    @functools.partial(
        pl.kernel,
        compiler_params=cp,
        mesh=plsc.VectorSubcoreMesh(core_axis_name="c", subcore_axis_name="s"),
        out_type=jax.ShapeDtypeStruct((E,), jnp.float32),
        scratch_types=[
            pltpu.VMEM((W,), jnp.int32),
            pltpu.VMEM((W,), jnp.int32),
            pltpu.VMEM((W, D), jnp.float32),
            pltpu.VMEM((W, D), jnp.float32),
            pltpu.VMEM((W,), jnp.float32),
            pltpu.SemaphoreType.DMA,
            pltpu.SemaphoreType.DMA,
        ],
    )
    def k(x_hbm, src_hbm, dst_hbm, out_hbm,
          sidx, didx, srows, drows, outv, sem_s, sem_d):
        wid = lax.axis_index("s") * NC + lax.axis_index("c")

        @pl.loop(0, steps)
        def _(t):
            j = wid + t * NW

            @pl.when(j < n_win)
            def _():
                base = j * W
                pltpu.sync_copy(src_hbm.at[pl.ds(base, W)], sidx)
                pltpu.sync_copy(dst_hbm.at[pl.ds(base, W)], didx)
                cs = pltpu.async_copy(x_hbm.at[sidx], srows, sem_s)
                cd = pltpu.async_copy(x_hbm.at[didx], drows, sem_d)
                cs.wait()
                cd.wait()

                @pl.loop(0, W // L)
                def _(g):
                    lane = lax.broadcasted_iota(jnp.int32, (L,), 0)
                    res = jnp.zeros((L,), jnp.float32)
                    for i in range(L):
                        e = g * L + i
                        acc = srows[e, pl.ds(0, L)] * drows[e, pl.ds(0, L)]
                        for c in range(1, D // L):
                            acc = acc + (srows[e, pl.ds(c * L, L)]
                                         * drows[e, pl.ds(c * L, L)])
                        res = jnp.where(lane == i, jnp.sum(acc), res)
                    outv[pl.ds(g * L, L)] = res

                pltpu.sync_copy(outv, out_hbm.at[pl.ds(base, W)])

    return k


@jax.jit
def kernel(x, edge_index):
    src = edge_index[0].astype(jnp.int32)
    dst = edge_index[1].astype(jnp.int32)
    return _build_sc_kernel(src.shape[0])(x, src, dst)


# speedup vs baseline: 2.6240x; 1.4290x over previous
import dataclasses
import functools

import jax
import jax.numpy as jnp
from jax import lax
from jax.experimental import pallas as pl
from jax.experimental.pallas import tpu as pltpu
from jax.experimental.pallas import tpu_sc as plsc

D = 256
L = 16
NC, NS = 2, 16
NW = NC * NS
W = 64


def _build_sc_kernel(E):
    PW = E // NW
    assert PW * NW == E and PW % 8 == 0
    NT = (PW + W - 1) // W
    last_off = PW - W

    cp = pltpu.CompilerParams()
    if "needs_layout_passes" in pltpu.CompilerParams.__dataclass_fields__:
        cp = dataclasses.replace(cp, needs_layout_passes=False)

    @functools.partial(
        pl.kernel,
        compiler_params=cp,
        mesh=plsc.VectorSubcoreMesh(core_axis_name="c", subcore_axis_name="s"),
        out_type=jax.ShapeDtypeStruct((E,), jnp.float32),
        scratch_types=[
            pltpu.VMEM((PW,), jnp.int32),
            pltpu.VMEM((PW,), jnp.int32),
            pltpu.VMEM((W, D), jnp.float32),
            pltpu.VMEM((W, D), jnp.float32),
            pltpu.VMEM((W, D), jnp.float32),
            pltpu.VMEM((W, D), jnp.float32),
            pltpu.VMEM((PW,), jnp.float32),
            pltpu.SemaphoreType.DMA,
            pltpu.SemaphoreType.DMA,
        ],
    )
    def k(x_hbm, src_hbm, dst_hbm, out_hbm,
          sidx, didx, srows0, drows0, srows1, drows1, outv, sem0, sem1):
        wid = lax.axis_index("s") * NC + lax.axis_index("c")
        base = wid * PW

        pltpu.sync_copy(src_hbm.at[pl.ds(base, PW)], sidx)
        pltpu.sync_copy(dst_hbm.at[pl.ds(base, PW)], didx)

        def off(t):
            return jnp.minimum(t * W, last_off)

        def issue(t, sb, db, sem):
            o = off(t)
            pltpu.async_copy(x_hbm.at[sidx.at[pl.ds(o, W)]], sb, sem)
            pltpu.async_copy(x_hbm.at[didx.at[pl.ds(o, W)]], db, sem)

        def drain(sb, db, sem):
            pltpu.make_async_copy(x_hbm.at[sidx.at[pl.ds(0, W)]], sb, sem).wait()
            pltpu.make_async_copy(x_hbm.at[didx.at[pl.ds(0, W)]], db, sem).wait()

        def compute(t, sb, db):
            o = off(t)

            @pl.loop(0, W // L)
            def _(g):
                lane = lax.broadcasted_iota(jnp.int32, (L,), 0)
                res = jnp.zeros((L,), jnp.float32)
                for i in range(L):
                    e = g * L + i
                    acc = sb[e, pl.ds(0, L)] * db[e, pl.ds(0, L)]
                    for c in range(1, D // L):
                        acc = acc + (sb[e, pl.ds(c * L, L)]
                                     * db[e, pl.ds(c * L, L)])
                    res = jnp.where(lane == i, jnp.sum(acc), res)
                outv[pl.ds(o + g * L, L)] = res

        issue(0, srows0, drows0, sem0)

        @pl.loop(0, NT, step=2)
        def _(t):
            @pl.when(t + 1 < NT)
            def _():
                issue(t + 1, srows1, drows1, sem1)

            drain(srows0, drows0, sem0)
            compute(t, srows0, drows0)

            @pl.when(t + 2 < NT)
            def _():
                issue(t + 2, srows0, drows0, sem0)

            @pl.when(t + 1 < NT)
            def _():
                drain(srows1, drows1, sem1)
                compute(t + 1, srows1, drows1)

        pltpu.sync_copy(outv, out_hbm.at[pl.ds(base, PW)])

    return k


@jax.jit
def kernel(x, edge_index):
    src = edge_index[0].astype(jnp.int32)
    dst = edge_index[1].astype(jnp.int32)
    return _build_sc_kernel(src.shape[0])(x, src, dst)
